# baseline (speedup 1.0000x reference)
import functools

import jax
import jax.numpy as jnp
from jax import lax
from jax.experimental import pallas as pl
from jax.experimental.pallas import tpu as pltpu
from jax.experimental.pallas import tpu_sc as plsc

B = 4096
L = 200
D = 64
NUM_ROWS = 1000

_info = plsc.get_sparse_core_info()
NC = _info.num_cores
NS = _info.num_subcores
NW = NC * NS

_mesh = plsc.VectorSubcoreMesh(core_axis_name="c", subcore_axis_name="s")


@functools.partial(
    pl.kernel,
    mesh=_mesh,
    out_type=jax.ShapeDtypeStruct((L, 8, NW, 8, 128), jnp.float32),
    scratch_types=[
        pltpu.VMEM((L, 128), jnp.int32),
        pltpu.VMEM((NUM_ROWS, D + 1), jnp.float32),
        pltpu.VMEM((2, 8, 8, 128), jnp.float32),
        pltpu.SemaphoreType.DMA,
        pltpu.SemaphoreType.DMA,
    ],
    compiler_params=pltpu.CompilerParams(
        use_tc_tiling_on_sc=False, needs_layout_passes=False),
)
def _gather_kernel(idx_hbm, table_hbm, out_hbm, idx_v, table_v, buf_v,
                   s0, s1):
    wid = lax.axis_index("s") * NC + lax.axis_index("c")
    ssem = (s0, s1)

    pltpu.sync_copy(table_hbm, table_v.at[:, pl.ds(0, D)])
    pltpu.sync_copy(idx_hbm.at[:, pl.ds(wid * 128, 128)], idx_v)

    def store(l, p):
        return pltpu.make_async_copy(
            buf_v.at[p], out_hbm.at[l, :, wid], ssem[p])

    def compute(l, p):
        idx16s = [idx_v[l, pl.ds(bg * 16, 16)] for bg in range(8)]

        @plsc.parallel_loop(0, D, unroll=8)
        def _d_loop(d):
            col = jnp.full((16,), 0, jnp.int32) + d
            for bg in range(8):
                v = plsc.load_gather(table_v, [idx16s[bg], col])
                buf_v[p, d // 8, d % 8, pl.ds(bg * 16, 16)] = v

    def body(g, carry):
        for p in range(2):
            l = g * 2 + p

            @pl.when(g > 0)
            def _wait_prev():
                store(l - 2, p).wait()

            compute(l, p)
            store(l, p).start()
        return carry

    lax.fori_loop(0, L // 2, body, 0)
    for p in range(2):
        store(L - 2 + p, p).wait()


def kernel(visit_segments, embedding_table):
    idx_t = visit_segments.T
    out5 = _gather_kernel(idx_t, embedding_table)
    return out5.transpose(2, 4, 0, 1, 3).reshape(B, L, D)


# speedup vs baseline: 1.2575x; 1.2575x over previous
import functools

import jax
import jax.numpy as jnp
from jax import lax
from jax.experimental import pallas as pl
from jax.experimental.pallas import tpu as pltpu
from jax.experimental.pallas import tpu_sc as plsc

B = 4096
L = 200
D = 64
NUM_ROWS = 1000

_info = plsc.get_sparse_core_info()
NC = _info.num_cores
NS = _info.num_subcores
NW = NC * NS

_mesh = plsc.VectorSubcoreMesh(core_axis_name="c", subcore_axis_name="s")


@functools.partial(
    pl.kernel,
    mesh=_mesh,
    out_type=jax.ShapeDtypeStruct((L, 8, NW, 8, 128), jnp.float32),
    scratch_types=[
        pltpu.VMEM((L // 8, 8, 128), jnp.int32),
        pltpu.VMEM((NUM_ROWS, D + 1), jnp.float32),
        pltpu.VMEM((2, 8, 8, 128), jnp.float32),
        pltpu.SemaphoreType.DMA,
        pltpu.SemaphoreType.DMA,
    ],
    compiler_params=pltpu.CompilerParams(
        use_tc_tiling_on_sc=False, needs_layout_passes=False),
)
def _gather_kernel(idx_hbm, table_hbm, out_hbm, idx_v, table_v, buf_v,
                   s0, s1):
    wid = lax.axis_index("s") * NC + lax.axis_index("c")
    ssem = (s0, s1)

    pltpu.sync_copy(table_hbm, table_v.at[:, pl.ds(0, D)])
    pltpu.sync_copy(idx_hbm.at[:, wid], idx_v)

    def store(l, p):
        return pltpu.make_async_copy(
            buf_v.at[p], out_hbm.at[l, :, wid], ssem[p])

    def compute(l, p):
        idx16s = [idx_v[l // 8, l % 8, pl.ds(bg * 16, 16)]
                  for bg in range(8)]

        @plsc.parallel_loop(0, D, unroll=4)
        def _d_loop(d):
            col = jnp.full((16,), 0, jnp.int32) + d
            for bg in range(8):
                v = plsc.load_gather(table_v, [idx16s[bg], col])
                buf_v[p, d // 8, d % 8, pl.ds(bg * 16, 16)] = v

    def body(g, carry):
        for p in range(2):
            l = g * 2 + p

            @pl.when(g > 0)
            def _wait_prev():
                store(l - 2, p).wait()

            compute(l, p)
            store(l, p).start()
        return carry

    lax.fori_loop(0, L // 2, body, 0)
    for p in range(2):
        store(L - 2 + p, p).wait()


def kernel(visit_segments, embedding_table):
    idx4 = visit_segments.reshape(NW, 128, L // 8, 8).transpose(2, 0, 3, 1)
    out5 = _gather_kernel(idx4, embedding_table)
    return out5.transpose(2, 4, 0, 1, 3).reshape(B, L, D)
